# baseline (speedup 1.0000x reference)
import functools

import jax
import jax.numpy as jnp
from jax import lax
from jax.experimental import pallas as pl
from jax.experimental.pallas import tpu as pltpu
from jax.experimental.pallas import tpu_sc as plsc

NUM_CORES = 2
NUM_SUBCORES = 16
NUM_WORKERS = NUM_CORES * NUM_SUBCORES
BATCH = 16384
DIM = 32
NUM_ROWS = 1000000
FULL_COLS = 7812
TAIL = 64
BASE_COLS = FULL_COLS // NUM_WORKERS
EXTRA_W = FULL_COLS - BASE_COLS * NUM_WORKERS
WIN = 4
WINL = WIN * 128
NWIN = BASE_COLS // WIN
CHUNK = 128
N_CHUNKS = (BATCH // NUM_WORKERS) // CHUNK
B_PER_W = BATCH // NUM_WORKERS


def _make_linearize():
    mesh = plsc.VectorSubcoreMesh(core_axis_name="c", subcore_axis_name="s")

    @functools.partial(
        pl.kernel,
        mesh=mesh,
        out_type=jax.ShapeDtypeStruct((NUM_ROWS * DIM,), jnp.float32),
        scratch_types=[
            pltpu.VMEM((DIM, WINL), jnp.float32),
            pltpu.VMEM((DIM, WINL), jnp.float32),
            pltpu.VMEM((WINL * DIM,), jnp.float32),
            pltpu.VMEM((WINL * DIM,), jnp.float32),
            pltpu.VMEM((DIM, 128), jnp.float32),
            pltpu.VMEM((128 * DIM,), jnp.float32),
            pltpu.SemaphoreType.DMA,
            pltpu.SemaphoreType.DMA,
            pltpu.SemaphoreType.DMA,
            pltpu.SemaphoreType.DMA,
            pltpu.SemaphoreType.DMA,
        ],
        compiler_params=pltpu.CompilerParams(needs_layout_passes=False),
    )
    def lin_kernel(table_hbm, lin_hbm, win0, win1, flat0, flat1,
                   xwin_v, xflat_v, isem0, isem1, osem0, osem1, xsem):
        wid = lax.axis_index("s") * NUM_CORES + lax.axis_index("c")
        c0 = wid * BASE_COLS + lax.min(wid, EXTRA_W)
        lane0 = c0 * 128

        lane_iota32 = lax.iota(jnp.int32, 16) * DIM

        def in_copy(g, win, isem):
            col = lane0 + g * WINL
            return pltpu.make_async_copy(
                table_hbm.at[:, pl.ds(pl.multiple_of(col, 128), WINL)],
                win,
                isem,
            )

        def out_copy(g, flat, osem):
            col = lane0 + g * WINL
            return pltpu.make_async_copy(
                flat,
                lin_hbm.at[pl.ds(col * DIM, WINL * DIM)],
                osem,
            )

        iota16 = lax.iota(jnp.int32, 16)
        c_vecs = [
            jnp.bitwise_and(c0 + iota16, DIM - 1) for c0 in range(DIM)
        ]

        def transpose_win(win, flat):
            @plsc.parallel_loop(0, WIN, step=1, unroll=2)
            def _(t):
                for vv in range(8):
                    l_vec = t * 128 + vv * 16 + iota16
                    l32 = l_vec * DIM
                    for c0 in range(DIM):
                        cv = c_vecs[c0]
                        vals = plsc.load_gather(win, [cv, l_vec])
                        plsc.store_scatter(flat, [l32 + cv], vals)

        in_copy(0, win0, isem0).start()

        def body(k, carry):
            g0 = k * 2
            g1 = g0 + 1

            @pl.when(g1 < NWIN)
            def _():
                in_copy(g1, win1, isem1).start()

            in_copy(g0, win0, isem0).wait()

            @pl.when(g0 >= 2)
            def _():
                out_copy(g0 - 2, flat0, osem0).wait()

            transpose_win(win0, flat0)
            out_copy(g0, flat0, osem0).start()

            @pl.when(g1 < NWIN)
            def _():
                @pl.when(g1 + 1 < NWIN)
                def _():
                    in_copy(g1 + 1, win0, isem0).start()

                in_copy(g1, win1, isem1).wait()

                @pl.when(g1 >= 2)
                def _():
                    out_copy(g1 - 2, flat1, osem1).wait()

                transpose_win(win1, flat1)
                out_copy(g1, flat1, osem1).start()

            return carry

        lax.fori_loop(0, (NWIN + 1) // 2, body, 0, unroll=False)

        out_copy(NWIN - 2, flat1, osem1).wait()
        out_copy(NWIN - 1, flat0, osem0).wait()

        @pl.when(wid < EXTRA_W)
        def _():
            xcol = c0 * 128 + NWIN * WINL
            xin = pltpu.make_async_copy(
                table_hbm.at[:, pl.ds(pl.multiple_of(xcol, 128), 128)],
                xwin_v,
                xsem,
            )
            xin.start()
            xin.wait()

            for vv in range(8):
                l_vec = vv * 16 + iota16
                l32 = l_vec * DIM
                for cc in range(DIM):
                    cv = c_vecs[cc]
                    vals = plsc.load_gather(xwin_v, [cv, l_vec])
                    plsc.store_scatter(xflat_v, [l32 + cv], vals)
            xout = pltpu.make_async_copy(
                xflat_v, lin_hbm.at[pl.ds(xcol * DIM, 128 * DIM)], xsem
            )
            xout.start()
            xout.wait()

    return lin_kernel


def _make_gather():
    mesh = plsc.VectorSubcoreMesh(core_axis_name="c", subcore_axis_name="s")

    @functools.partial(
        pl.kernel,
        mesh=mesh,
        out_type=jax.ShapeDtypeStruct((BATCH, DIM), jnp.float32),
        scratch_types=[
            pltpu.VMEM((N_CHUNKS, CHUNK), jnp.int32),
            pltpu.VMEM((B_PER_W, DIM), jnp.float32),
            pltpu.SemaphoreType.DMA,
        ],
        compiler_params=pltpu.CompilerParams(use_tc_tiling_on_sc=False),
    )
    def gather_kernel(idx_hbm, lin_hbm, out_hbm, idx_v, rows_v, sem):
        wid = lax.axis_index("s") * NUM_CORES + lax.axis_index("c")
        base = wid * B_PER_W
        pltpu.sync_copy(idx_hbm.at[wid], idx_v)
        copies = []
        for j in range(N_CHUNKS):
            copies.append(
                pltpu.async_copy(
                    lin_hbm.at[idx_v.at[j]],
                    rows_v.at[pl.ds(j * CHUNK, CHUNK)],
                    sem,
                )
            )
        for c in copies:
            c.wait()
        pltpu.sync_copy(rows_v, out_hbm.at[pl.ds(base, B_PER_W)])

    return gather_kernel


_linearize = _make_linearize()
_sc_gather = _make_gather()


def kernel(indices, weight):
    lin = _linearize(weight.T)
    tail = weight[FULL_COLS * 128:, :].reshape(-1)
    lin = lax.dynamic_update_slice(lin, tail, (FULL_COLS * 128 * DIM,))
    lin2d = lin.reshape(NUM_ROWS, DIM)
    idx3 = indices.astype(jnp.int32).reshape(NUM_WORKERS, N_CHUNKS, CHUNK)
    return _sc_gather(idx3, lin2d)


# speedup vs baseline: 3.1975x; 3.1975x over previous
import functools

import jax
import jax.numpy as jnp
from jax import lax
from jax.experimental import pallas as pl
from jax.experimental.pallas import tpu as pltpu
from jax.experimental.pallas import tpu_sc as plsc

NUM_CORES = 2
NUM_SUBCORES = 16
NUM_WORKERS = NUM_CORES * NUM_SUBCORES
BATCH = 16384
DIM = 32
NUM_ROWS = 1000000
FULL_COLS = 7812
TAIL = 64
BASE_COLS = FULL_COLS // NUM_WORKERS
EXTRA_W = FULL_COLS - BASE_COLS * NUM_WORKERS
WIN = 4
WINL = WIN * 128
NWIN = BASE_COLS // WIN
CHUNK = 128
N_CHUNKS = (BATCH // NUM_WORKERS) // CHUNK
B_PER_W = BATCH // NUM_WORKERS


def _make_linearize():
    mesh = plsc.VectorSubcoreMesh(core_axis_name="c", subcore_axis_name="s")

    @functools.partial(
        pl.kernel,
        mesh=mesh,
        out_type=jax.ShapeDtypeStruct((NUM_ROWS * DIM,), jnp.float32),
        scratch_types=[
            pltpu.VMEM((DIM, WINL), jnp.float32),
            pltpu.VMEM((DIM, WINL), jnp.float32),
            pltpu.VMEM((WINL * DIM,), jnp.float32),
            pltpu.VMEM((WINL * DIM,), jnp.float32),
            pltpu.VMEM((DIM, 128), jnp.float32),
            pltpu.VMEM((128 * DIM,), jnp.float32),
            pltpu.SemaphoreType.DMA,
            pltpu.SemaphoreType.DMA,
            pltpu.SemaphoreType.DMA,
            pltpu.SemaphoreType.DMA,
            pltpu.SemaphoreType.DMA,
        ],
        compiler_params=pltpu.CompilerParams(needs_layout_passes=False),
    )
    def lin_kernel(table_hbm, lin_hbm, win0, win1, flat0, flat1,
                   xwin_v, xflat_v, isem0, isem1, osem0, osem1, xsem):
        wid = lax.axis_index("s") * NUM_CORES + lax.axis_index("c")
        c0 = wid * BASE_COLS + lax.min(wid, EXTRA_W)
        lane0 = c0 * 128

        lane_iota32 = lax.iota(jnp.int32, 16) * DIM

        def in_copy(g, win, isem):
            col = lane0 + g * WINL
            return pltpu.make_async_copy(
                table_hbm.at[:, pl.ds(pl.multiple_of(col, 128), WINL)],
                win,
                isem,
            )

        def out_copy(g, flat, osem):
            col = lane0 + g * WINL
            return pltpu.make_async_copy(
                flat,
                lin_hbm.at[pl.ds(col * DIM, WINL * DIM)],
                osem,
            )

        iota16 = lax.iota(jnp.int32, 16)
        c_vecs = [
            jnp.bitwise_and(c0 + iota16, DIM - 1) for c0 in range(DIM)
        ]

        def transpose_win(win, flat):
            def over_t(t, carry):
                for vv in range(8):
                    l_vec = t * 128 + vv * 16 + iota16
                    l32 = l_vec * DIM
                    for c0 in range(0, DIM, 8):
                        vals = [
                            plsc.load_gather(win, [c_vecs[c0 + k], l_vec])
                            for k in range(8)
                        ]
                        for k in range(8):
                            plsc.store_scatter(
                                flat, [l32 + c_vecs[c0 + k]], vals[k]
                            )
                return carry

            lax.fori_loop(0, WIN, over_t, 0, unroll=False)

        in_copy(0, win0, isem0).start()

        def body(k, carry):
            g0 = k * 2
            g1 = g0 + 1

            @pl.when(g1 < NWIN)
            def _():
                in_copy(g1, win1, isem1).start()

            in_copy(g0, win0, isem0).wait()

            @pl.when(g0 >= 2)
            def _():
                out_copy(g0 - 2, flat0, osem0).wait()

            transpose_win(win0, flat0)
            out_copy(g0, flat0, osem0).start()

            @pl.when(g1 < NWIN)
            def _():
                @pl.when(g1 + 1 < NWIN)
                def _():
                    in_copy(g1 + 1, win0, isem0).start()

                in_copy(g1, win1, isem1).wait()

                @pl.when(g1 >= 2)
                def _():
                    out_copy(g1 - 2, flat1, osem1).wait()

                transpose_win(win1, flat1)
                out_copy(g1, flat1, osem1).start()

            return carry

        lax.fori_loop(0, (NWIN + 1) // 2, body, 0, unroll=False)

        out_copy(NWIN - 2, flat1, osem1).wait()
        out_copy(NWIN - 1, flat0, osem0).wait()

        @pl.when(wid < EXTRA_W)
        def _():
            xcol = c0 * 128 + NWIN * WINL
            xin = pltpu.make_async_copy(
                table_hbm.at[:, pl.ds(pl.multiple_of(xcol, 128), 128)],
                xwin_v,
                xsem,
            )
            xin.start()
            xin.wait()

            for vv in range(8):
                l_vec = vv * 16 + iota16
                l32 = l_vec * DIM
                for cc in range(DIM):
                    cv = c_vecs[cc]
                    vals = plsc.load_gather(xwin_v, [cv, l_vec])
                    plsc.store_scatter(xflat_v, [l32 + cv], vals)
            xout = pltpu.make_async_copy(
                xflat_v, lin_hbm.at[pl.ds(xcol * DIM, 128 * DIM)], xsem
            )
            xout.start()
            xout.wait()

    return lin_kernel


def _make_gather():
    mesh = plsc.VectorSubcoreMesh(core_axis_name="c", subcore_axis_name="s")

    @functools.partial(
        pl.kernel,
        mesh=mesh,
        out_type=jax.ShapeDtypeStruct((BATCH, DIM), jnp.float32),
        scratch_types=[
            pltpu.VMEM((N_CHUNKS, CHUNK), jnp.int32),
            pltpu.VMEM((B_PER_W, DIM), jnp.float32),
            pltpu.SemaphoreType.DMA,
        ],
        compiler_params=pltpu.CompilerParams(use_tc_tiling_on_sc=False),
    )
    def gather_kernel(idx_hbm, lin_hbm, out_hbm, idx_v, rows_v, sem):
        wid = lax.axis_index("s") * NUM_CORES + lax.axis_index("c")
        base = wid * B_PER_W
        pltpu.sync_copy(idx_hbm.at[wid], idx_v)
        copies = []
        for j in range(N_CHUNKS):
            copies.append(
                pltpu.async_copy(
                    lin_hbm.at[idx_v.at[j]],
                    rows_v.at[pl.ds(j * CHUNK, CHUNK)],
                    sem,
                )
            )
        for c in copies:
            c.wait()
        pltpu.sync_copy(rows_v, out_hbm.at[pl.ds(base, B_PER_W)])

    return gather_kernel


_linearize = _make_linearize()
_sc_gather = _make_gather()


def kernel(indices, weight):
    lin = _linearize(weight.T)
    tail = weight[FULL_COLS * 128:, :].reshape(-1)
    lin = lax.dynamic_update_slice(lin, tail, (FULL_COLS * 128 * DIM,))
    lin2d = lin.reshape(NUM_ROWS, DIM)
    idx3 = indices.astype(jnp.int32).reshape(NUM_WORKERS, N_CHUNKS, CHUNK)
    return _sc_gather(idx3, lin2d)
